# baseline (speedup 1.0000x reference)
import functools
import jax
import jax.numpy as jnp
from jax import lax
from jax.experimental import pallas as pl
from jax.experimental.pallas import tpu as pltpu, tpu_sc as plsc

N_FIELDS = 26
N_EMB = 100
BATCH = 16384
OUT_W = N_FIELDS * N_EMB
NW = 32
ROWS_PER_W = BATCH // NW
R = 16
NCH = ROWS_PER_W // R

_mesh = plsc.VectorSubcoreMesh(core_axis_name="c", subcore_axis_name="s")


@functools.partial(
    pl.kernel,
    mesh=_mesh,
    out_type=jax.ShapeDtypeStruct((BATCH, OUT_W), jnp.int32),
    scratch_types=[
        pltpu.VMEM((R, OUT_W), jnp.int32),
        pltpu.VMEM((R, OUT_W), jnp.int32),
        pltpu.VMEM((R, N_FIELDS), jnp.int32),
        pltpu.VMEM((R, N_FIELDS), jnp.int32),
        pltpu.SemaphoreType.DMA,
        pltpu.SemaphoreType.DMA,
    ],
    compiler_params=pltpu.CompilerParams(
        use_tc_tiling_on_sc=False, needs_layout_passes=False),
)
def _sc_onehot(idx_hbm, zeros_hbm, out_hbm, buf0, buf1, idxb0, idxb1,
               sem0, sem1):
    wid = lax.axis_index("s") * 2 + lax.axis_index("c")
    sub_base = wid * ROWS_PER_W

    off0 = lax.iota(jnp.int32, 16) * N_EMB
    off1 = off0 + 10 * N_EMB
    ones = jnp.full((16,), 1, jnp.int32)
    zeros = jnp.full((16,), 0, jnp.int32)

    def scatter_rows(buf, idxb, value):
        for r in range(R):
            rows = jnp.full((16,), r, jnp.int32)
            v0 = idxb[r, pl.ds(0, 16)] + off0
            v1 = idxb[r, pl.ds(10, 16)] + off1
            plsc.store_scatter(buf, [rows, v0], value)
            plsc.store_scatter(buf, [rows, v1], value)

    def process_chunk(base, buf, idxb, sem, clear):
        if clear:
            scatter_rows(buf, idxb, zeros)
        pltpu.sync_copy(idx_hbm.at[pl.ds(base, R)], idxb)
        scatter_rows(buf, idxb, ones)
        pltpu.make_async_copy(buf, out_hbm.at[pl.ds(base, R)], sem).start()

    pltpu.sync_copy(zeros_hbm, buf0)
    pltpu.sync_copy(zeros_hbm, buf1)
    process_chunk(sub_base, buf0, idxb0, sem0, clear=False)
    process_chunk(sub_base + R, buf1, idxb1, sem1, clear=False)

    def body(g, _):
        base = sub_base + 2 * g * R
        pltpu.make_async_copy(buf0, out_hbm.at[pl.ds(sub_base, R)], sem0).wait()
        process_chunk(base, buf0, idxb0, sem0, clear=True)
        pltpu.make_async_copy(buf1, out_hbm.at[pl.ds(sub_base, R)], sem1).wait()
        process_chunk(base + R, buf1, idxb1, sem1, clear=True)
        return _

    lax.fori_loop(1, NCH // 2, body, None)
    pltpu.make_async_copy(buf0, out_hbm.at[pl.ds(sub_base, R)], sem0).wait()
    pltpu.make_async_copy(buf1, out_hbm.at[pl.ds(sub_base, R)], sem1).wait()


def kernel(index_list):
    zeros_hbm = jnp.zeros((R, OUT_W), jnp.int32)
    return _sc_onehot(index_list, zeros_hbm)


# speedup vs baseline: 1.6480x; 1.6480x over previous
import functools
import jax
import jax.numpy as jnp
from jax import lax
from jax.experimental import pallas as pl
from jax.experimental.pallas import tpu as pltpu, tpu_sc as plsc

N_FIELDS = 26
N_EMB = 100
BATCH = 16384
OUT_W = N_FIELDS * N_EMB
NW = 32
ROWS_PER_W = BATCH // NW
R = 16
NCH = ROWS_PER_W // R

_mesh = plsc.VectorSubcoreMesh(core_axis_name="c", subcore_axis_name="s")


@functools.partial(
    pl.kernel,
    mesh=_mesh,
    out_type=jax.ShapeDtypeStruct((BATCH, OUT_W), jnp.int32),
    scratch_types=[
        pltpu.VMEM((R, OUT_W), jnp.int32),
        pltpu.VMEM((R, OUT_W), jnp.int32),
        pltpu.VMEM((R, N_FIELDS), jnp.int32),
        pltpu.VMEM((R, N_FIELDS), jnp.int32),
        pltpu.SemaphoreType.DMA,
        pltpu.SemaphoreType.DMA,
    ],
    compiler_params=pltpu.CompilerParams(
        use_tc_tiling_on_sc=True, needs_layout_passes=False),
)
def _sc_onehot(idx_hbm, zeros_hbm, out_hbm, buf0, buf1, idxb0, idxb1,
               sem0, sem1):
    wid = lax.axis_index("s") * 2 + lax.axis_index("c")
    sub_base = wid * ROWS_PER_W

    off0 = lax.iota(jnp.int32, 16) * N_EMB
    off1 = off0 + 10 * N_EMB
    ones = jnp.full((16,), 1, jnp.int32)
    zeros = jnp.full((16,), 0, jnp.int32)

    def scatter_rows(buf, idxb, value):
        for r in range(R):
            rows = jnp.full((16,), r, jnp.int32)
            v0 = idxb[r, pl.ds(0, 16)] + off0
            v1 = idxb[r, pl.ds(10, 16)] + off1
            plsc.store_scatter(buf, [rows, v0], value)
            plsc.store_scatter(buf, [rows, v1], value)

    def process_chunk(base, buf, idxb, sem, clear):
        if clear:
            scatter_rows(buf, idxb, zeros)
        pltpu.sync_copy(idx_hbm.at[pl.ds(base, R)], idxb)
        scatter_rows(buf, idxb, ones)
        pltpu.make_async_copy(buf, out_hbm.at[pl.ds(base, R)], sem).start()

    pltpu.sync_copy(zeros_hbm, buf0)
    pltpu.sync_copy(zeros_hbm, buf1)
    process_chunk(sub_base, buf0, idxb0, sem0, clear=False)
    process_chunk(sub_base + R, buf1, idxb1, sem1, clear=False)

    def body(g, _):
        base = sub_base + 2 * g * R
        pltpu.make_async_copy(buf0, out_hbm.at[pl.ds(sub_base, R)], sem0).wait()
        process_chunk(base, buf0, idxb0, sem0, clear=True)
        pltpu.make_async_copy(buf1, out_hbm.at[pl.ds(sub_base, R)], sem1).wait()
        process_chunk(base + R, buf1, idxb1, sem1, clear=True)
        return _

    lax.fori_loop(1, NCH // 2, body, None)
    pltpu.make_async_copy(buf0, out_hbm.at[pl.ds(sub_base, R)], sem0).wait()
    pltpu.make_async_copy(buf1, out_hbm.at[pl.ds(sub_base, R)], sem1).wait()


def kernel(index_list):
    zeros_hbm = jnp.zeros((R, OUT_W), jnp.int32)
    return _sc_onehot(index_list, zeros_hbm)


# speedup vs baseline: 4.8835x; 2.9633x over previous
import functools
import jax
import jax.numpy as jnp
from jax import lax
from jax.experimental import pallas as pl
from jax.experimental.pallas import tpu as pltpu, tpu_sc as plsc

N_FIELDS = 26
N_EMB = 100
BATCH = 16384
OUT_W = N_FIELDS * N_EMB
NW = 32
COLS_PER_W = BATCH // NW
CC = 128
RC = 2 * N_EMB
NCOL = COLS_PER_W // CC
NROW = OUT_W // RC
NCH = NCOL * NROW
NG = CC // 16

_mesh = plsc.VectorSubcoreMesh(core_axis_name="c", subcore_axis_name="s")


@functools.partial(
    pl.kernel,
    mesh=_mesh,
    out_type=jax.ShapeDtypeStruct((OUT_W, BATCH), jnp.int32),
    scratch_types=[
        pltpu.VMEM((RC, CC), jnp.int32),
        pltpu.VMEM((RC, CC), jnp.int32),
        pltpu.VMEM((2 * NG, 16), jnp.int32),
        pltpu.VMEM((2 * NG, 16), jnp.int32),
        pltpu.VMEM((N_FIELDS, CC), jnp.int32),
        pltpu.SemaphoreType.DMA,
        pltpu.SemaphoreType.DMA,
    ],
    compiler_params=pltpu.CompilerParams(
        use_tc_tiling_on_sc=True, needs_layout_passes=False),
)
def _sc_onehot_t(idxt_hbm, zeros_hbm, out_hbm, buf0, buf1, pos0, pos1,
                 idxc, sem0, sem1):
    wid = lax.axis_index("s") * 2 + lax.axis_index("c")
    col_base = wid * COLS_PER_W

    lane = lax.iota(jnp.int32, 16)
    ones = jnp.full((16,), 1, jnp.int32)
    zeros16 = jnp.full((16,), 0, jnp.int32)

    def process_chunk(m, buf, pos):
        for f in range(2):
            for g in range(NG):
                cols = lane + 16 * g
                old_rows = pos[2 * g + f, :]
                new_rows = idxc[2 * m + f, pl.ds(16 * g, 16)] + f * N_EMB
                plsc.store_scatter(buf, [old_rows, cols], zeros16,
                                   mask=old_rows != new_rows)
                plsc.store_scatter(buf, [new_rows, cols], ones)
                pos[2 * g + f, :] = new_rows

    pltpu.sync_copy(zeros_hbm, buf0)
    pltpu.sync_copy(zeros_hbm, buf1)
    for g in range(NG):
        for f in range(2):
            band = jnp.full((16,), f * N_EMB, jnp.int32)
            pos0[2 * g + f, :] = band
            pos1[2 * g + f, :] = band
    pltpu.sync_copy(idxt_hbm.at[:, pl.ds(col_base, CC)], idxc)
    process_chunk(0, buf0, pos0)
    pltpu.make_async_copy(
        buf0, out_hbm.at[pl.ds(0, RC), pl.ds(col_base, CC)], sem0).start()
    process_chunk(1, buf1, pos1)
    pltpu.make_async_copy(
        buf1, out_hbm.at[pl.ds(RC, RC), pl.ds(col_base, CC)], sem1).start()

    def body(t, _):
        c = t // NROW
        m = t - c * NROW

        @pl.when(m == 0)
        def _():
            pltpu.sync_copy(
                idxt_hbm.at[:, pl.ds(col_base + c * CC, CC)], idxc)

        col_off = col_base + c * CC

        @pl.when(t % 2 == 0)
        def _():
            pltpu.make_async_copy(
                buf0, out_hbm.at[pl.ds(0, RC), pl.ds(col_base, CC)],
                sem0).wait()
            process_chunk(m, buf0, pos0)
            pltpu.make_async_copy(
                buf0, out_hbm.at[pl.ds(m * RC, RC), pl.ds(col_off, CC)],
                sem0).start()

        @pl.when(t % 2 == 1)
        def _():
            pltpu.make_async_copy(
                buf1, out_hbm.at[pl.ds(0, RC), pl.ds(col_base, CC)],
                sem1).wait()
            process_chunk(m, buf1, pos1)
            pltpu.make_async_copy(
                buf1, out_hbm.at[pl.ds(m * RC, RC), pl.ds(col_off, CC)],
                sem1).start()
        return _

    lax.fori_loop(2, NCH, body, None)
    pltpu.make_async_copy(
        buf0, out_hbm.at[pl.ds(0, RC), pl.ds(col_base, CC)], sem0).wait()
    pltpu.make_async_copy(
        buf1, out_hbm.at[pl.ds(0, RC), pl.ds(col_base, CC)], sem1).wait()


def kernel(index_list):
    idxT = index_list.T
    zeros_hbm = jnp.zeros((RC, CC), jnp.int32)
    outT = _sc_onehot_t(idxT, zeros_hbm)
    return outT.T
